# Initial kernel scaffold; baseline (speedup 1.0000x reference)
import jax
import jax.numpy as jnp
from jax.experimental import pallas as pl


def kernel(x, emb_proton, emb_neutron, W, b):
    raise NotImplementedError("write your pallas kernel here")


# speedup vs baseline: 1.3415x; 1.3415x over previous
import functools

import jax
import jax.numpy as jnp
from jax import lax
from jax.experimental import pallas as pl
from jax.experimental.pallas import tpu as pltpu
from jax.experimental.pallas import tpu_sc as plsc

_L = 16
_NC = 2
_NS = 16
_NW = _NC * _NS
_ICHUNK = 128


@functools.lru_cache(maxsize=None)
def _build(B, H):
    bpw = B // _NW
    n_ichunks = bpw // _ICHUNK
    n_wreg = 2 * H // _L
    mesh = plsc.VectorSubcoreMesh(core_axis_name="c", subcore_axis_name="s")

    @functools.partial(
        pl.kernel,
        mesh=mesh,
        out_type=jax.ShapeDtypeStruct((B,), jnp.float32),
        compiler_params=pltpu.CompilerParams(
            needs_layout_passes=False, use_tc_tiling_on_sc=False
        ),
        scratch_types=[
            pltpu.VMEM((2 * bpw,), jnp.int32),
            pltpu.VMEM((bpw,), jnp.int32),
            pltpu.VMEM((bpw,), jnp.int32),
            pltpu.VMEM((bpw, H), jnp.float32),
            pltpu.VMEM((bpw, H), jnp.float32),
            pltpu.VMEM((2 * H,), jnp.float32),
            pltpu.VMEM((_L,), jnp.float32),
            pltpu.VMEM((bpw,), jnp.float32),
            pltpu.SemaphoreType.DMA,
        ],
    )
    def sc_kernel(x_hbm, pt_hbm, nt_hbm, w_hbm, b_hbm, out_hbm,
                  xv, piv, niv, prv, nrv, wv, bv, ov, sem):
        wid = lax.axis_index("s") * _NC + lax.axis_index("c")
        base = wid * bpw

        pltpu.sync_copy(x_hbm.at[pl.ds(2 * base, 2 * bpw)], xv)
        pltpu.sync_copy(w_hbm, wv)
        pltpu.sync_copy(b_hbm, bv)

        iota = lax.iota(jnp.int32, _L)

        def deint(g, carry):
            strided = 2 * (g * _L + iota)
            piv[pl.ds(g * _L, _L)] = plsc.load_gather(xv, [strided])
            niv[pl.ds(g * _L, _L)] = plsc.load_gather(xv, [strided + 1])
            return carry
        lax.fori_loop(0, bpw // _L, deint, 0)

        copies = []
        for c in range(n_ichunks):
            sl = pl.ds(c * _ICHUNK, _ICHUNK)
            copies.append(pltpu.async_copy(pt_hbm.at[piv.at[sl]], prv.at[sl], sem))
            copies.append(pltpu.async_copy(nt_hbm.at[niv.at[sl]], nrv.at[sl], sem))
        for cp in copies:
            cp.wait()

        wregs = [wv[pl.ds(k * _L, _L)] for k in range(n_wreg)]
        b_vec = bv[...]

        def row_body(g, carry):
            out_vec = b_vec
            for u in range(_L):
                r = g * _L + u
                acc = prv[r, pl.ds(0, _L)] * wregs[0]
                for k in range(1, n_wreg // 2):
                    acc += prv[r, pl.ds(k * _L, _L)] * wregs[k]
                for k in range(n_wreg // 2):
                    acc += nrv[r, pl.ds(k * _L, _L)] * wregs[n_wreg // 2 + k]
                out_vec = jnp.where(iota == u, out_vec + jnp.sum(acc), out_vec)
            ov[pl.ds(g * _L, _L)] = out_vec
            return carry
        lax.fori_loop(0, bpw // _L, row_body, 0)

        pltpu.sync_copy(ov, out_hbm.at[pl.ds(base, bpw)])

    return sc_kernel


def kernel(x, emb_proton, emb_neutron, W, b):
    B = x.shape[0]
    H = emb_proton.shape[1]
    b_vec = jnp.broadcast_to(b.reshape(()), (_L,)).astype(jnp.float32)
    out = _build(B, H)(x.reshape(-1), emb_proton, emb_neutron, W.reshape(-1), b_vec)
    return out.reshape(B, 1)
